# baseline (speedup 1.0000x reference)
import jax
import jax.numpy as jnp
from jax.experimental import pallas as pl

NB = 8
P = 2048
K = 16
DG = 2
C = 256
C_PTS = 64
ADD_C = 64
EPS = 1e-5


def _bn(x, g, b, axes):
    m = jnp.mean(x, axis=axes, keepdims=True)
    v = jnp.var(x, axis=axes, keepdims=True)
    sh = [1] * x.ndim
    sh[1] = x.shape[1]
    return (x - m) / jnp.sqrt(v + EPS) * g.reshape(sh) + b.reshape(sh)


def _copy_kernel(x_ref, o_ref):
    o_ref[...] = x_ref[...]


def kernel(pts, fts_prev, W_fc1, b_fc1, g1, be1, W_fc2, b_fc2, g2, be2, W_c1, b_c1, g3, be3, W_c2, b_c2):
    rA = jnp.sum(pts * pts, axis=2, keepdims=True)
    dist = rA - 2.0 * jnp.einsum('npd,nqd->npq', pts, pts) + jnp.swapaxes(rA, 1, 2)
    _, idx = jax.lax.top_k(-dist, K)

    pts_f = jnp.swapaxes(pts, 1, 2)
    group = jax.vmap(lambda f, i: f[:, i])
    pts_grouped = group(pts_f, idx)
    fts_grouped = group(fts_prev, idx)

    fts = pts_grouped - pts_f[:, :, :, None]
    fts = jnp.transpose(fts, (0, 2, 3, 1))
    fts = jnp.einsum('npkd,hd->npkh', fts, W_fc1) + b_fc1
    fts = jnp.transpose(fts, (0, 3, 1, 2))
    fts = jax.nn.relu(_bn(fts, g1, be1, (0, 2, 3)))
    fts = jnp.transpose(fts, (0, 2, 3, 1))
    fts = jnp.einsum('npkh,ch->npkc', fts, W_fc2) + b_fc2
    fts = jnp.transpose(fts, (0, 3, 1, 2))
    fts = jax.nn.relu(_bn(fts, g2, be2, (0, 2, 3)))

    fts = jnp.concatenate([fts, fts_grouped], axis=1)
    cin = fts.shape[1]
    fts = fts.reshape(NB, cin, P, DG, K // DG).max(axis=-1)
    xg = fts.reshape(NB, DG, cin // DG, P, DG)
    wg = W_c1.reshape(DG, C // DG, cin // DG, DG)
    out = jnp.einsum('ngipw,goiw->ngop', xg, wg).reshape(NB, C, P) + b_c1[None, :, None]
    out = jax.nn.relu(_bn(out, g3, be3, (0, 2)))
    out = jnp.einsum('ncp,oc->nop', out, W_c2) + b_c2[None, :, None]
    out = pl.pallas_call(
        _copy_kernel,
        out_shape=jax.ShapeDtypeStruct(out.shape, out.dtype),
    )(out)
    fts_out = jnp.concatenate([fts_prev, out], axis=1)
    return (pts, fts_out)


# speedup vs baseline: 1.4563x; 1.4563x over previous
import functools

import jax
import jax.numpy as jnp
from jax import lax
from jax.experimental import pallas as pl
from jax.experimental.pallas import tpu as pltpu
from jax.experimental.pallas import tpu_sc as plsc

NB = 8
P = 2048
K = 16
DG = 2
C = 256
C_PTS = 64
ADD_C = 64
EPS = 1e-5

NW = 32
WPB = NW // NB
RW = P // WPB
NV = P // 16

_INF = float("inf")


def _sc_mesh():
    return plsc.VectorSubcoreMesh(core_axis_name="c", subcore_axis_name="s")


def _topk_body(xs_hbm, ys_hbm, zs_hbm, idx_hbm, xs_s, ys_s, zs_s, ra_s, dist_s, cd_s, ci_s, idxo_s, sem):
    wid = lax.axis_index("s") * 2 + lax.axis_index("c")
    n = wid // WPB
    q0 = (wid % WPB) * RW

    pltpu.sync_copy(xs_hbm.at[pl.ds(n * P, P)], xs_s)
    pltpu.sync_copy(ys_hbm.at[pl.ds(n * P, P)], ys_s)
    pltpu.sync_copy(zs_hbm.at[pl.ds(n * P, P)], zs_s)

    def _rnd(v):
        u = plsc.bitcast(v, jnp.uint32)
        u = (u + jnp.uint32(0x7FFF) + ((u >> jnp.uint32(16)) & jnp.uint32(1)))
        u = u & jnp.uint32(0xFFFF0000)
        return plsc.bitcast(u, jnp.float32)

    def _ra(j, _):
        s = pl.ds(j * 16, 16)
        x = xs_s[s]
        y = ys_s[s]
        z = zs_s[s]
        ra_s[s] = x * x + y * y + z * z
        xs_s[s] = _rnd(x)
        ys_s[s] = _rnd(y)
        zs_s[s] = _rnd(z)
        return 0
    lax.fori_loop(0, NV, _ra, 0, unroll=4)

    iota16 = lax.iota(jnp.int32, 16)

    def _row(r, _):
        q = q0 + r
        qsel = jnp.full((16,), q, jnp.int32)
        qx = plsc.load_gather(xs_s, [qsel])
        qy = plsc.load_gather(ys_s, [qsel])
        qz = plsc.load_gather(zs_s, [qsel])
        qr = plsc.load_gather(ra_s, [qsel])

        def _pa(j, acc):
            s = pl.ds(j * 16, 16)
            x = xs_s[s]
            y = ys_s[s]
            z = zs_s[s]
            ra = ra_s[s]
            d = (qr - 2.0 * (qx * x + qy * y + qz * z)) + ra
            dist_s[s] = d
            return jnp.minimum(acc, d)
        acc = lax.fori_loop(0, NV, _pa, jnp.full((16,), _INF), unroll=4)

        t = jnp.max(acc)
        tv = jnp.full((16,), t)

        def _pb(j, carry):
            cnt, idxv = carry
            d = dist_s[pl.ds(j * 16, 16)]
            m = d <= tv
            plsc.store_compressed(cd_s.at[pl.ds(cnt, 16)], d, mask=m)
            plsc.store_compressed(ci_s.at[pl.ds(cnt, 16)], idxv, mask=m)
            pc = plsc.all_reduce_population_count(m)
            return cnt + jnp.max(pc), idxv + 16
        cnt, _ = lax.fori_loop(0, NV, _pb, (jnp.int32(0), iota16), unroll=4)

        cnt_v = jnp.full((16,), cnt, jnp.int32)

        def _pm(jj, carry):
            best, bidx = carry
            b = jj * 16
            dv = cd_s[pl.ds(b, 16)]
            iv = ci_s[pl.ds(b, 16)]
            valid = (jnp.full((16,), b, jnp.int32) + iota16) < cnt_v
            dv = jnp.where(valid, dv, _INF)
            sd, si = plsc.sort_key_val(dv, iv)
            rd = lax.rev(sd, (0,))
            ri = lax.rev(si, (0,))
            take = best <= rd
            nd = jnp.where(take, best, rd)
            ni = jnp.where(take, bidx, ri)
            nd, ni = plsc.sort_key_val(nd, ni)
            return nd, ni
        best0 = jnp.full((16,), _INF)
        bidx0 = jnp.zeros((16,), jnp.int32)
        _, bidx = lax.fori_loop(0, (cnt + 15) // 16, _pm, (best0, bidx0))

        idxo_s[pl.ds(r * K, K)] = bidx
        return 0

    lax.fori_loop(0, RW, _row, 0)
    pltpu.sync_copy(idxo_s, idx_hbm.at[pl.ds((n * P + q0) * K, RW * K)])


@functools.partial(jax.jit, static_argnames=())
def _sc_topk(xs, ys, zs):
    f = functools.partial(
        pl.kernel,
        out_type=jax.ShapeDtypeStruct((NB * P * K,), jnp.int32),
        mesh=_sc_mesh(),
        compiler_params=pltpu.CompilerParams(needs_layout_passes=False),
        scratch_types=[
            pltpu.VMEM((P,), jnp.float32),
            pltpu.VMEM((P,), jnp.float32),
            pltpu.VMEM((P,), jnp.float32),
            pltpu.VMEM((P,), jnp.float32),
            pltpu.VMEM((P,), jnp.float32),
            pltpu.VMEM((P + 16,), jnp.float32),
            pltpu.VMEM((P + 16,), jnp.int32),
            pltpu.VMEM((RW * K,), jnp.int32),
            pltpu.SemaphoreType.DMA,
        ],
    )(_topk_body)
    return f(xs, ys, zs).reshape(NB, P, K)


def _bn(x, g, b, axes):
    m = jnp.mean(x, axis=axes, keepdims=True)
    v = jnp.var(x, axis=axes, keepdims=True)
    sh = [1] * x.ndim
    sh[1] = x.shape[1]
    return (x - m) / jnp.sqrt(v + EPS) * g.reshape(sh) + b.reshape(sh)


def kernel(pts, fts_prev, W_fc1, b_fc1, g1, be1, W_fc2, b_fc2, g2, be2, W_c1, b_c1, g3, be3, W_c2, b_c2):
    ptsT = jnp.swapaxes(pts, 1, 2)
    xs = ptsT[:, 0].reshape(-1)
    ys = ptsT[:, 1].reshape(-1)
    zs = ptsT[:, 2].reshape(-1)
    idx = _sc_topk(xs, ys, zs)

    pts_f = ptsT
    group = jax.vmap(lambda f, i: f[:, i])
    pts_grouped = group(pts_f, idx)
    fts_grouped = group(fts_prev, idx)

    fts = pts_grouped - pts_f[:, :, :, None]
    fts = jnp.transpose(fts, (0, 2, 3, 1))
    fts = jnp.einsum('npkd,hd->npkh', fts, W_fc1) + b_fc1
    fts = jnp.transpose(fts, (0, 3, 1, 2))
    fts = jax.nn.relu(_bn(fts, g1, be1, (0, 2, 3)))
    fts = jnp.transpose(fts, (0, 2, 3, 1))
    fts = jnp.einsum('npkh,ch->npkc', fts, W_fc2) + b_fc2
    fts = jnp.transpose(fts, (0, 3, 1, 2))
    fts = jax.nn.relu(_bn(fts, g2, be2, (0, 2, 3)))

    fts = jnp.concatenate([fts, fts_grouped], axis=1)
    cin = fts.shape[1]
    fts = fts.reshape(NB, cin, P, DG, K // DG).max(axis=-1)
    xg = fts.reshape(NB, DG, cin // DG, P, DG)
    wg = W_c1.reshape(DG, C // DG, cin // DG, DG)
    out = jnp.einsum('ngipw,goiw->ngop', xg, wg).reshape(NB, C, P) + b_c1[None, :, None]
    out = jax.nn.relu(_bn(out, g3, be3, (0, 2)))
    out = jnp.einsum('ncp,oc->nop', out, W_c2) + b_c2[None, :, None]
    fts_out = jnp.concatenate([fts_prev, out], axis=1)
    return (pts, fts_out)


# speedup vs baseline: 6.6458x; 4.5635x over previous
import functools

import jax
import jax.numpy as jnp
from jax import lax
from jax.experimental import pallas as pl
from jax.experimental.pallas import tpu as pltpu
from jax.experimental.pallas import tpu_sc as plsc

NB = 8
P = 2048
K = 16
DG = 2
C = 256
C_PTS = 64
ADD_C = 64
EPS = 1e-5
H = C_PTS // 2

NW = 32
WPB = NW // NB
RW = P // WPB
NV = P // 16
GR = 8
NG = RW // GR

_INF = float("inf")


def _sc_body(xs_hbm, ys_hbm, zs_hbm, ftsT_hbm,
             relx_hbm, rely_hbm, relz_hbm, gmax_hbm, stats_hbm,
             xs_s, ys_s, zs_s, ra_s, xo_s, yo_s, zo_s,
             dist_s, cd_s, ci_s,
             relx_s, rely_s, relz_s, gm_s, gidx_s, grows_s, st_s,
             sem, gsem):
    wid = lax.axis_index("s") * 2 + lax.axis_index("c")
    n = wid // WPB
    q0 = (wid % WPB) * RW

    pltpu.sync_copy(xs_hbm.at[pl.ds(n * P, P)], xo_s)
    pltpu.sync_copy(ys_hbm.at[pl.ds(n * P, P)], yo_s)
    pltpu.sync_copy(zs_hbm.at[pl.ds(n * P, P)], zo_s)

    def _rnd(v):
        u = plsc.bitcast(v, jnp.uint32)
        u = (u + jnp.uint32(0x7FFF) + ((u >> jnp.uint32(16)) & jnp.uint32(1)))
        u = u & jnp.uint32(0xFFFF0000)
        return plsc.bitcast(u, jnp.float32)

    def _ra(j, _):
        s = pl.ds(j * 16, 16)
        x = xo_s[s]
        y = yo_s[s]
        z = zo_s[s]
        ra_s[s] = x * x + y * y + z * z
        xs_s[s] = _rnd(x)
        ys_s[s] = _rnd(y)
        zs_s[s] = _rnd(z)
        return 0
    lax.fori_loop(0, NV, _ra, 0, unroll=4)

    iota16 = lax.iota(jnp.int32, 16)
    npv = jnp.full((16,), n * P, jnp.int32)

    def _topk_row(q):
        qsel = jnp.full((16,), q, jnp.int32)
        qx = plsc.load_gather(xs_s, [qsel])
        qy = plsc.load_gather(ys_s, [qsel])
        qz = plsc.load_gather(zs_s, [qsel])
        qr = plsc.load_gather(ra_s, [qsel])

        def _pa(j, acc):
            s = pl.ds(j * 16, 16)
            x = xs_s[s]
            y = ys_s[s]
            z = zs_s[s]
            ra = ra_s[s]
            d = (qr - 2.0 * (qx * x + qy * y + qz * z)) + ra
            dist_s[s] = d
            return jnp.minimum(acc, d)
        acc = lax.fori_loop(0, NV, _pa, jnp.full((16,), _INF), unroll=4)
        t = jnp.max(acc)
        tv = jnp.full((16,), t)

        def _pb(j, carry):
            cnt, idxv = carry
            d = dist_s[pl.ds(j * 16, 16)]
            m = d <= tv
            plsc.store_compressed(cd_s.at[pl.ds(cnt, 16)], d, mask=m)
            plsc.store_compressed(ci_s.at[pl.ds(cnt, 16)], idxv, mask=m)
            pc = plsc.all_reduce_population_count(m)
            return cnt + jnp.max(pc), idxv + 16
        cnt, _ = lax.fori_loop(0, NV, _pb, (jnp.int32(0), iota16), unroll=4)
        cnt_v = jnp.full((16,), cnt, jnp.int32)

        def _pm(jj, carry):
            best, bidx = carry
            b = jj * 16
            dv = cd_s[pl.ds(b, 16)]
            iv = ci_s[pl.ds(b, 16)]
            valid = (jnp.full((16,), b, jnp.int32) + iota16) < cnt_v
            dv = jnp.where(valid, dv, _INF)
            sd, si = plsc.sort_key_val(dv, iv)
            rd = lax.rev(sd, (0,))
            ri = lax.rev(si, (0,))
            take = best <= rd
            nd = jnp.where(take, best, rd)
            ni = jnp.where(take, bidx, ri)
            nd, ni = plsc.sort_key_val(nd, ni)
            return nd, ni
        best0 = jnp.full((16,), _INF)
        bidx0 = jnp.zeros((16,), jnp.int32)
        _, bidx = lax.fori_loop(0, (cnt + 15) // 16, _pm, (best0, bidx0))
        return bidx, qsel

    def _grp(g, stat):
        sx, sy, sz, sxx, syy, szz, sxy, sxz, syz = stat
        r0 = g * GR
        for rr in range(GR):
            r = r0 + rr
            bidx, qsel = _topk_row(q0 + r)
            gidx_s[pl.ds(rr * 16, 16)] = bidx + npv
            qx = plsc.load_gather(xo_s, [qsel])
            qy = plsc.load_gather(yo_s, [qsel])
            qz = plsc.load_gather(zo_s, [qsel])
            rx = plsc.load_gather(xo_s, [bidx]) - qx
            ry = plsc.load_gather(yo_s, [bidx]) - qy
            rz = plsc.load_gather(zo_s, [bidx]) - qz
            relx_s[pl.ds(r * 16, 16)] = rx
            rely_s[pl.ds(r * 16, 16)] = ry
            relz_s[pl.ds(r * 16, 16)] = rz
            sx += rx
            sy += ry
            sz += rz
            sxx += rx * rx
            syy += ry * ry
            szz += rz * rz
            sxy += rx * ry
            sxz += rx * rz
            syz += ry * rz
        pltpu.async_copy(ftsT_hbm.at[gidx_s], grows_s, gsem).wait()
        for rr in range(GR):
            for g2 in range(2):
                for c4 in range(4):
                    s = pl.ds(c4 * 16, 16)
                    m = grows_s[rr * 16 + g2 * 8, s]
                    for kk in range(1, 8):
                        m = jnp.maximum(m, grows_s[rr * 16 + g2 * 8 + kk, s])
                    gm_s[pl.ds(rr * 128 + g2 * 64 + c4 * 16, 16)] = m
        pltpu.sync_copy(
            gm_s, gmax_hbm.at[pl.ds((n * P + q0 + r0) * 128, GR * 128)])
        return (sx, sy, sz, sxx, syy, szz, sxy, sxz, syz)

    z16 = jnp.zeros((16,), jnp.float32)
    stat = lax.fori_loop(0, NG, _grp, (z16,) * 9)
    for i in range(9):
        st_s[pl.ds(i * 16, 16)] = stat[i]
    pltpu.sync_copy(relx_s, relx_hbm.at[pl.ds((n * P + q0) * K, RW * K)])
    pltpu.sync_copy(rely_s, rely_hbm.at[pl.ds((n * P + q0) * K, RW * K)])
    pltpu.sync_copy(relz_s, relz_hbm.at[pl.ds((n * P + q0) * K, RW * K)])
    pltpu.sync_copy(st_s, stats_hbm.at[pl.ds(wid * 144, 144)])


def _sc_stage(xs, ys, zs, ftsT):
    f = functools.partial(
        pl.kernel,
        out_type=(
            jax.ShapeDtypeStruct((NB * P * K,), jnp.float32),
            jax.ShapeDtypeStruct((NB * P * K,), jnp.float32),
            jax.ShapeDtypeStruct((NB * P * K,), jnp.float32),
            jax.ShapeDtypeStruct((NB * P * 2 * C_PTS,), jnp.float32),
            jax.ShapeDtypeStruct((NW * 9 * 16,), jnp.float32),
        ),
        mesh=plsc.VectorSubcoreMesh(core_axis_name="c", subcore_axis_name="s"),
        compiler_params=pltpu.CompilerParams(needs_layout_passes=False),
        scratch_types=[
            pltpu.VMEM((P,), jnp.float32),
            pltpu.VMEM((P,), jnp.float32),
            pltpu.VMEM((P,), jnp.float32),
            pltpu.VMEM((P,), jnp.float32),
            pltpu.VMEM((P,), jnp.float32),
            pltpu.VMEM((P,), jnp.float32),
            pltpu.VMEM((P,), jnp.float32),
            pltpu.VMEM((P,), jnp.float32),
            pltpu.VMEM((P + 16,), jnp.float32),
            pltpu.VMEM((P + 16,), jnp.int32),
            pltpu.VMEM((RW * K,), jnp.float32),
            pltpu.VMEM((RW * K,), jnp.float32),
            pltpu.VMEM((RW * K,), jnp.float32),
            pltpu.VMEM((GR * 128,), jnp.float32),
            pltpu.VMEM((GR * K,), jnp.int32),
            pltpu.VMEM((GR * K, 128), jnp.float32),
            pltpu.VMEM((9 * 16,), jnp.float32),
            pltpu.SemaphoreType.DMA,
            pltpu.SemaphoreType.DMA,
        ],
    )(_sc_body)
    return f(xs, ys, zs, ftsT)


TPB = 512
TPC = 1024
TPD = 1024


def _kb_body(relx_ref, rely_ref, relz_ref, stats_ref, w1_ref, b1_ref,
             g1_ref, be1_ref, w2_ref, b2_ref,
             pooled_ref, s2_ref, acc_ref):
    ni = pl.program_id(0)
    pi = pl.program_id(1)
    first = jnp.logical_and(ni == 0, pi == 0)
    last = jnp.logical_and(ni == NB - 1, pi == (P // TPB) - 1)

    @pl.when(first)
    def _():
        acc_ref[...] = jnp.zeros_like(acc_ref)

    s = jnp.sum(stats_ref[...], axis=(0, 2)) / float(NB * P * K)
    mu = s[0:3]
    m2 = jnp.stack([
        jnp.stack([s[3], s[6], s[7]]),
        jnp.stack([s[6], s[4], s[8]]),
        jnp.stack([s[7], s[8], s[5]]),
    ])
    cov = m2 - mu[:, None] * mu[None, :]
    w1 = w1_ref[...]
    m1 = w1 @ mu + b1_ref[...]
    v1 = jnp.sum((w1 @ cov) * w1, axis=1)
    sc1 = g1_ref[...] * jax.lax.rsqrt(v1 + EPS)
    sh1 = be1_ref[...] - m1 * sc1

    rx = relx_ref[0]
    ry = rely_ref[0]
    rz = relz_ref[0]
    z1 = (rx[:, :, None] * w1[:, 0][None, None, :]
          + ry[:, :, None] * w1[:, 1][None, None, :]
          + rz[:, :, None] * w1[:, 2][None, None, :]
          + b1_ref[...][None, None, :])
    a1 = jnp.maximum(z1 * sc1[None, None, :] + sh1[None, None, :], 0.0)
    z2 = jax.lax.dot_general(
        a1.reshape(TPB * K, H), w2_ref[...],
        (((1,), (1,)), ((), ())),
        preferred_element_type=jnp.float32) + b2_ref[...][None, :]
    acc_ref[0, :] += jnp.sum(z2, axis=0)
    acc_ref[1, :] += jnp.sum(z2 * z2, axis=0)
    zp = z2.reshape(TPB, 2, K // 2, C_PTS).max(axis=2)
    pooled_ref[0] = zp

    @pl.when(last)
    def _():
        s2_ref[...] = acc_ref[...]


def _kb(relx, rely, relz, stats, w1, b1, g1, be1, w2, b2):
    grid = (NB, P // TPB)
    return pl.pallas_call(
        _kb_body,
        grid=grid,
        in_specs=[
            pl.BlockSpec((1, TPB, K), lambda ni, pi: (ni, pi, 0)),
            pl.BlockSpec((1, TPB, K), lambda ni, pi: (ni, pi, 0)),
            pl.BlockSpec((1, TPB, K), lambda ni, pi: (ni, pi, 0)),
            pl.BlockSpec((NW, 9, 16), lambda ni, pi: (0, 0, 0)),
            pl.BlockSpec((H, 3), lambda ni, pi: (0, 0)),
            pl.BlockSpec((H,), lambda ni, pi: (0,)),
            pl.BlockSpec((H,), lambda ni, pi: (0,)),
            pl.BlockSpec((H,), lambda ni, pi: (0,)),
            pl.BlockSpec((C_PTS, H), lambda ni, pi: (0, 0)),
            pl.BlockSpec((C_PTS,), lambda ni, pi: (0,)),
        ],
        out_specs=[
            pl.BlockSpec((1, TPB, 2, C_PTS), lambda ni, pi: (ni, pi, 0, 0)),
            pl.BlockSpec((2, C_PTS), lambda ni, pi: (0, 0)),
        ],
        out_shape=[
            jax.ShapeDtypeStruct((NB, P, 2, C_PTS), jnp.float32),
            jax.ShapeDtypeStruct((2, C_PTS), jnp.float32),
        ],
        scratch_shapes=[pltpu.VMEM((2, C_PTS), jnp.float32)],
    )(relx, rely, relz, stats, w1, b1, g1, be1, w2, b2)


def _kc_body(pooled_ref, gmax_ref, s2_ref, wm_ref, bc1_ref, g2_ref, be2_ref,
             y_ref, s3_ref, acc_ref):
    ni = pl.program_id(0)
    pi = pl.program_id(1)
    first = jnp.logical_and(ni == 0, pi == 0)
    last = jnp.logical_and(ni == NB - 1, pi == (P // TPC) - 1)

    @pl.when(first)
    def _():
        acc_ref[...] = jnp.zeros_like(acc_ref)

    n2 = float(NB * P * K)
    m2 = s2_ref[0] / n2
    v2 = s2_ref[1] / n2 - m2 * m2
    sc2 = g2_ref[...] * jax.lax.rsqrt(v2 + EPS)
    sh2 = be2_ref[...] - m2 * sc2

    x1 = jnp.maximum(pooled_ref[0] * sc2[None, None, :]
                     + sh2[None, None, :], 0.0).reshape(TPC, 2 * C_PTS)
    x2 = gmax_ref[0].reshape(TPC, 2 * C_PTS)
    y = (jax.lax.dot_general(x1, wm_ref[0], (((1,), (0,)), ((), ())),
                             preferred_element_type=jnp.float32)
         + jax.lax.dot_general(x2, wm_ref[1], (((1,), (0,)), ((), ())),
                               preferred_element_type=jnp.float32)
         + bc1_ref[...][None, :])
    acc_ref[0, :] += jnp.sum(y, axis=0)
    acc_ref[1, :] += jnp.sum(y * y, axis=0)
    y_ref[0] = y

    @pl.when(last)
    def _():
        s3_ref[...] = acc_ref[...]


def _kc(pooled, gmax, s2, wm, bc1, g2, be2):
    grid = (NB, P // TPC)
    return pl.pallas_call(
        _kc_body,
        grid=grid,
        in_specs=[
            pl.BlockSpec((1, TPC, 2, C_PTS), lambda ni, pi: (ni, pi, 0, 0)),
            pl.BlockSpec((1, TPC, 2, C_PTS), lambda ni, pi: (ni, pi, 0, 0)),
            pl.BlockSpec((2, C_PTS), lambda ni, pi: (0, 0)),
            pl.BlockSpec((2, 2 * C_PTS, C), lambda ni, pi: (0, 0, 0)),
            pl.BlockSpec((C,), lambda ni, pi: (0,)),
            pl.BlockSpec((C_PTS,), lambda ni, pi: (0,)),
            pl.BlockSpec((C_PTS,), lambda ni, pi: (0,)),
        ],
        out_specs=[
            pl.BlockSpec((1, TPC, C), lambda ni, pi: (ni, pi, 0)),
            pl.BlockSpec((2, C), lambda ni, pi: (0, 0)),
        ],
        out_shape=[
            jax.ShapeDtypeStruct((NB, P, C), jnp.float32),
            jax.ShapeDtypeStruct((2, C), jnp.float32),
        ],
        scratch_shapes=[pltpu.VMEM((2, C), jnp.float32)],
    )(pooled, gmax, s2, wm, bc1, g2, be2)


def _kd_body(y_ref, s3_ref, w2_ref, bc2_ref, g3_ref, be3_ref, out_ref):
    n3 = float(NB * P)
    m3 = s3_ref[0] / n3
    v3 = s3_ref[1] / n3 - m3 * m3
    sc3 = g3_ref[...] * jax.lax.rsqrt(v3 + EPS)
    sh3 = be3_ref[...] - m3 * sc3
    a = jnp.maximum(y_ref[0] * sc3[None, :] + sh3[None, :], 0.0)
    o = jax.lax.dot_general(w2_ref[...], a, (((1,), (1,)), ((), ())),
                            preferred_element_type=jnp.float32)
    out_ref[0] = o + bc2_ref[...][:, None]


def _kd(y, s3, w2, bc2, g3, be3):
    grid = (NB, P // TPD)
    return pl.pallas_call(
        _kd_body,
        grid=grid,
        in_specs=[
            pl.BlockSpec((1, TPD, C), lambda ni, pi: (ni, pi, 0)),
            pl.BlockSpec((2, C), lambda ni, pi: (0, 0)),
            pl.BlockSpec((C // 4, C), lambda ni, pi: (0, 0)),
            pl.BlockSpec((C // 4,), lambda ni, pi: (0,)),
            pl.BlockSpec((C,), lambda ni, pi: (0,)),
            pl.BlockSpec((C,), lambda ni, pi: (0,)),
        ],
        out_specs=pl.BlockSpec((1, C // 4, TPD), lambda ni, pi: (ni, 0, pi)),
        out_shape=jax.ShapeDtypeStruct((NB, C // 4, P), jnp.float32),
    )(y, s3, w2, bc2, g3, be3)


def kernel(pts, fts_prev, W_fc1, b_fc1, g1, be1, W_fc2, b_fc2, g2, be2, W_c1, b_c1, g3, be3, W_c2, b_c2):
    ptsT = jnp.swapaxes(pts, 1, 2)
    xs = ptsT[:, 0].reshape(-1)
    ys = ptsT[:, 1].reshape(-1)
    zs = ptsT[:, 2].reshape(-1)
    ftsT = jnp.swapaxes(fts_prev, 1, 2).reshape(NB * P, ADD_C)
    ftsT = jnp.pad(ftsT, ((0, 0), (0, 128 - ADD_C)))

    relx, rely, relz, gmax, stats = _sc_stage(xs, ys, zs, ftsT)
    relx = relx.reshape(NB, P, K)
    rely = rely.reshape(NB, P, K)
    relz = relz.reshape(NB, P, K)
    gmax = gmax.reshape(NB, P, 2, C_PTS)
    stats = stats.reshape(NW, 9, 16)

    pooled, s2 = _kb(relx, rely, relz, stats, W_fc1, b_fc1, g1, be1, W_fc2, b_fc2)

    wm = W_c1.reshape(DG, C // DG, 2 * C_PTS // DG, DG)
    wm = jnp.transpose(wm, (0, 3, 2, 1)).reshape(DG, 2 * C_PTS, C // DG)
    wm = jnp.concatenate([
        jnp.concatenate([wm[0], jnp.zeros_like(wm[0])], axis=1)[None],
        jnp.concatenate([jnp.zeros_like(wm[1]), wm[1]], axis=1)[None],
    ], axis=0)

    y, s3 = _kc(pooled, gmax, s2, wm, b_c1, g2, be2)
    out = _kd(y, s3, W_c2, b_c2, g3, be3)
    fts_out = jnp.concatenate([fts_prev, out], axis=1)
    return (pts, fts_out)


# speedup vs baseline: 7.3388x; 1.1043x over previous
import functools

import jax
import jax.numpy as jnp
from jax import lax
from jax.experimental import pallas as pl
from jax.experimental.pallas import tpu as pltpu
from jax.experimental.pallas import tpu_sc as plsc

NB = 8
P = 2048
K = 16
DG = 2
C = 256
C_PTS = 64
ADD_C = 64
EPS = 1e-5
H = C_PTS // 2

NW = 32
WPB = NW // NB
RW = P // WPB
NV = P // 16
GR = 8
NG = RW // GR

_INF = float("inf")


def _sc_body(xs_hbm, ys_hbm, zs_hbm, ftsT_hbm,
             relx_hbm, rely_hbm, relz_hbm, gmax_hbm, stats_hbm,
             xs_s, ys_s, zs_s, ra_s, xo_s, yo_s, zo_s,
             dist_s, cd_s, ci_s,
             relx_s, rely_s, relz_s, gm_s, gidx_s, grows_s, st_s,
             sem, gsem):
    wid = lax.axis_index("s") * 2 + lax.axis_index("c")
    n = wid // WPB
    q0 = (wid % WPB) * RW

    pltpu.sync_copy(xs_hbm.at[pl.ds(n * P, P)], xo_s)
    pltpu.sync_copy(ys_hbm.at[pl.ds(n * P, P)], yo_s)
    pltpu.sync_copy(zs_hbm.at[pl.ds(n * P, P)], zo_s)

    def _rnd(v):
        u = plsc.bitcast(v, jnp.uint32)
        u = (u + jnp.uint32(0x7FFF) + ((u >> jnp.uint32(16)) & jnp.uint32(1)))
        u = u & jnp.uint32(0xFFFF0000)
        return plsc.bitcast(u, jnp.float32)

    def _ra(j, _):
        s = pl.ds(j * 16, 16)
        x = xo_s[s]
        y = yo_s[s]
        z = zo_s[s]
        ra_s[s] = x * x + y * y + z * z
        xs_s[s] = _rnd(x)
        ys_s[s] = _rnd(y)
        zs_s[s] = _rnd(z)
        return 0
    lax.fori_loop(0, NV, _ra, 0, unroll=4)

    iota16 = lax.iota(jnp.int32, 16)
    npv = jnp.full((16,), n * P, jnp.int32)

    def _topk_row(q):
        qsel = jnp.full((16,), q, jnp.int32)
        qx = plsc.load_gather(xs_s, [qsel])
        qy = plsc.load_gather(ys_s, [qsel])
        qz = plsc.load_gather(zs_s, [qsel])
        qr = plsc.load_gather(ra_s, [qsel])

        def _pa(j, acc):
            s = pl.ds(j * 16, 16)
            x = xs_s[s]
            y = ys_s[s]
            z = zs_s[s]
            ra = ra_s[s]
            d = (qr - 2.0 * (qx * x + qy * y + qz * z)) + ra
            dist_s[s] = d
            return jnp.minimum(acc, d)
        acc = lax.fori_loop(0, NV, _pa, jnp.full((16,), _INF), unroll=4)
        t = jnp.max(acc)
        tv = jnp.full((16,), t)

        def _pb(j, carry):
            cnt, idxv = carry
            d = dist_s[pl.ds(j * 16, 16)]
            m = d <= tv
            plsc.store_compressed(cd_s.at[pl.ds(cnt, 16)], d, mask=m)
            plsc.store_compressed(ci_s.at[pl.ds(cnt, 16)], idxv, mask=m)
            pc = plsc.all_reduce_population_count(m)
            return cnt + pc[0], idxv + 16
        cnt, _ = lax.fori_loop(0, NV, _pb, (jnp.int32(0), iota16), unroll=4)
        cnt_v = jnp.full((16,), cnt, jnp.int32)

        def _pm(jj, carry):
            best, bidx = carry
            b = jj * 16
            dv = cd_s[pl.ds(b, 16)]
            iv = ci_s[pl.ds(b, 16)]
            valid = (jnp.full((16,), b, jnp.int32) + iota16) < cnt_v
            dv = jnp.where(valid, dv, _INF)
            sd, si = plsc.sort_key_val(dv, iv)
            rd = lax.rev(sd, (0,))
            ri = lax.rev(si, (0,))
            take = best <= rd
            nd = jnp.where(take, best, rd)
            ni = jnp.where(take, bidx, ri)
            nd, ni = plsc.sort_key_val(nd, ni)
            return nd, ni
        best0 = jnp.full((16,), _INF)
        bidx0 = jnp.zeros((16,), jnp.int32)
        _, bidx = lax.fori_loop(0, (cnt + 15) // 16, _pm, (best0, bidx0))
        return bidx, qsel

    def _grp(g, stat):
        sx, sy, sz, sxx, syy, szz, sxy, sxz, syz = stat
        r0 = g * GR
        for rr in range(GR):
            r = r0 + rr
            bidx, qsel = _topk_row(q0 + r)
            gidx_s[pl.ds(rr * 16, 16)] = bidx + npv
            qx = plsc.load_gather(xo_s, [qsel])
            qy = plsc.load_gather(yo_s, [qsel])
            qz = plsc.load_gather(zo_s, [qsel])
            rx = plsc.load_gather(xo_s, [bidx]) - qx
            ry = plsc.load_gather(yo_s, [bidx]) - qy
            rz = plsc.load_gather(zo_s, [bidx]) - qz
            relx_s[pl.ds(r * 16, 16)] = rx
            rely_s[pl.ds(r * 16, 16)] = ry
            relz_s[pl.ds(r * 16, 16)] = rz
            sx += rx
            sy += ry
            sz += rz
            sxx += rx * rx
            syy += ry * ry
            szz += rz * rz
            sxy += rx * ry
            sxz += rx * rz
            syz += ry * rz
        pltpu.async_copy(ftsT_hbm.at[gidx_s], grows_s, gsem).wait()
        for rr in range(GR):
            for g2 in range(2):
                for c4 in range(4):
                    s = pl.ds(c4 * 16, 16)
                    m = grows_s[rr * 16 + g2 * 8, s]
                    for kk in range(1, 8):
                        m = jnp.maximum(m, grows_s[rr * 16 + g2 * 8 + kk, s])
                    gm_s[pl.ds(rr * 128 + g2 * 64 + c4 * 16, 16)] = m
        pltpu.sync_copy(
            gm_s, gmax_hbm.at[pl.ds((n * P + q0 + r0) * 128, GR * 128)])
        return (sx, sy, sz, sxx, syy, szz, sxy, sxz, syz)

    z16 = jnp.zeros((16,), jnp.float32)
    stat = lax.fori_loop(0, NG, _grp, (z16,) * 9)
    for i in range(9):
        st_s[pl.ds(i * 16, 16)] = stat[i]
    pltpu.sync_copy(relx_s, relx_hbm.at[pl.ds((n * P + q0) * K, RW * K)])
    pltpu.sync_copy(rely_s, rely_hbm.at[pl.ds((n * P + q0) * K, RW * K)])
    pltpu.sync_copy(relz_s, relz_hbm.at[pl.ds((n * P + q0) * K, RW * K)])
    pltpu.sync_copy(st_s, stats_hbm.at[pl.ds(wid * 144, 144)])


def _sc_stage(xs, ys, zs, ftsT):
    f = functools.partial(
        pl.kernel,
        out_type=(
            jax.ShapeDtypeStruct((NB * P * K,), jnp.float32),
            jax.ShapeDtypeStruct((NB * P * K,), jnp.float32),
            jax.ShapeDtypeStruct((NB * P * K,), jnp.float32),
            jax.ShapeDtypeStruct((NB * P * 2 * C_PTS,), jnp.float32),
            jax.ShapeDtypeStruct((NW * 9 * 16,), jnp.float32),
        ),
        mesh=plsc.VectorSubcoreMesh(core_axis_name="c", subcore_axis_name="s"),
        compiler_params=pltpu.CompilerParams(needs_layout_passes=False),
        scratch_types=[
            pltpu.VMEM((P,), jnp.float32),
            pltpu.VMEM((P,), jnp.float32),
            pltpu.VMEM((P,), jnp.float32),
            pltpu.VMEM((P,), jnp.float32),
            pltpu.VMEM((P,), jnp.float32),
            pltpu.VMEM((P,), jnp.float32),
            pltpu.VMEM((P,), jnp.float32),
            pltpu.VMEM((P,), jnp.float32),
            pltpu.VMEM((P + 16,), jnp.float32),
            pltpu.VMEM((P + 16,), jnp.int32),
            pltpu.VMEM((RW * K,), jnp.float32),
            pltpu.VMEM((RW * K,), jnp.float32),
            pltpu.VMEM((RW * K,), jnp.float32),
            pltpu.VMEM((GR * 128,), jnp.float32),
            pltpu.VMEM((GR * K,), jnp.int32),
            pltpu.VMEM((GR * K, 128), jnp.float32),
            pltpu.VMEM((9 * 16,), jnp.float32),
            pltpu.SemaphoreType.DMA,
            pltpu.SemaphoreType.DMA,
        ],
    )(_sc_body)
    return f(xs, ys, zs, ftsT)


TPB = 512
TPC = 1024
TPD = 1024


def _kb_body(relx_ref, rely_ref, relz_ref, stats_ref, w1_ref, b1_ref,
             g1_ref, be1_ref, w2_ref, b2_ref,
             pooled_ref, s2_ref, acc_ref):
    ni = pl.program_id(0)
    pi = pl.program_id(1)
    first = jnp.logical_and(ni == 0, pi == 0)
    last = jnp.logical_and(ni == NB - 1, pi == (P // TPB) - 1)

    @pl.when(first)
    def _():
        acc_ref[...] = jnp.zeros_like(acc_ref)

    s = jnp.sum(stats_ref[...], axis=(0, 2)) / float(NB * P * K)
    mu = s[0:3]
    m2 = jnp.stack([
        jnp.stack([s[3], s[6], s[7]]),
        jnp.stack([s[6], s[4], s[8]]),
        jnp.stack([s[7], s[8], s[5]]),
    ])
    cov = m2 - mu[:, None] * mu[None, :]
    w1 = w1_ref[...]
    m1 = w1 @ mu + b1_ref[...]
    v1 = jnp.sum((w1 @ cov) * w1, axis=1)
    sc1 = g1_ref[...] * jax.lax.rsqrt(v1 + EPS)
    sh1 = be1_ref[...] - m1 * sc1

    rx = relx_ref[0]
    ry = rely_ref[0]
    rz = relz_ref[0]
    z1 = (rx[:, :, None] * w1[:, 0][None, None, :]
          + ry[:, :, None] * w1[:, 1][None, None, :]
          + rz[:, :, None] * w1[:, 2][None, None, :]
          + b1_ref[...][None, None, :])
    a1 = jnp.maximum(z1 * sc1[None, None, :] + sh1[None, None, :], 0.0)
    z2 = jax.lax.dot_general(
        a1.reshape(TPB * K, H), w2_ref[...],
        (((1,), (1,)), ((), ())),
        preferred_element_type=jnp.float32) + b2_ref[...][None, :]
    acc_ref[0, :] += jnp.sum(z2, axis=0)
    acc_ref[1, :] += jnp.sum(z2 * z2, axis=0)
    zp = z2.reshape(TPB, 2, K // 2, C_PTS).max(axis=2)
    pooled_ref[0] = zp

    @pl.when(last)
    def _():
        s2_ref[...] = acc_ref[...]


def _kb(relx, rely, relz, stats, w1, b1, g1, be1, w2, b2):
    grid = (NB, P // TPB)
    return pl.pallas_call(
        _kb_body,
        grid=grid,
        in_specs=[
            pl.BlockSpec((1, TPB, K), lambda ni, pi: (ni, pi, 0)),
            pl.BlockSpec((1, TPB, K), lambda ni, pi: (ni, pi, 0)),
            pl.BlockSpec((1, TPB, K), lambda ni, pi: (ni, pi, 0)),
            pl.BlockSpec((NW, 9, 16), lambda ni, pi: (0, 0, 0)),
            pl.BlockSpec((H, 3), lambda ni, pi: (0, 0)),
            pl.BlockSpec((H,), lambda ni, pi: (0,)),
            pl.BlockSpec((H,), lambda ni, pi: (0,)),
            pl.BlockSpec((H,), lambda ni, pi: (0,)),
            pl.BlockSpec((C_PTS, H), lambda ni, pi: (0, 0)),
            pl.BlockSpec((C_PTS,), lambda ni, pi: (0,)),
        ],
        out_specs=[
            pl.BlockSpec((1, TPB, 2, C_PTS), lambda ni, pi: (ni, pi, 0, 0)),
            pl.BlockSpec((2, C_PTS), lambda ni, pi: (0, 0)),
        ],
        out_shape=[
            jax.ShapeDtypeStruct((NB, P, 2, C_PTS), jnp.float32),
            jax.ShapeDtypeStruct((2, C_PTS), jnp.float32),
        ],
        scratch_shapes=[pltpu.VMEM((2, C_PTS), jnp.float32)],
    )(relx, rely, relz, stats, w1, b1, g1, be1, w2, b2)


def _kc_body(pooled_ref, gmax_ref, s2_ref, wm_ref, bc1_ref, g2_ref, be2_ref,
             y_ref, s3_ref, acc_ref):
    ni = pl.program_id(0)
    pi = pl.program_id(1)
    first = jnp.logical_and(ni == 0, pi == 0)
    last = jnp.logical_and(ni == NB - 1, pi == (P // TPC) - 1)

    @pl.when(first)
    def _():
        acc_ref[...] = jnp.zeros_like(acc_ref)

    n2 = float(NB * P * K)
    m2 = s2_ref[0] / n2
    v2 = s2_ref[1] / n2 - m2 * m2
    sc2 = g2_ref[...] * jax.lax.rsqrt(v2 + EPS)
    sh2 = be2_ref[...] - m2 * sc2

    x1 = jnp.maximum(pooled_ref[0] * sc2[None, None, :]
                     + sh2[None, None, :], 0.0).reshape(TPC, 2 * C_PTS)
    x2 = gmax_ref[0].reshape(TPC, 2 * C_PTS)
    y = (jax.lax.dot_general(x1, wm_ref[0], (((1,), (0,)), ((), ())),
                             preferred_element_type=jnp.float32)
         + jax.lax.dot_general(x2, wm_ref[1], (((1,), (0,)), ((), ())),
                               preferred_element_type=jnp.float32)
         + bc1_ref[...][None, :])
    acc_ref[0, :] += jnp.sum(y, axis=0)
    acc_ref[1, :] += jnp.sum(y * y, axis=0)
    y_ref[0] = y

    @pl.when(last)
    def _():
        s3_ref[...] = acc_ref[...]


def _kc(pooled, gmax, s2, wm, bc1, g2, be2):
    grid = (NB, P // TPC)
    return pl.pallas_call(
        _kc_body,
        grid=grid,
        in_specs=[
            pl.BlockSpec((1, TPC, 2, C_PTS), lambda ni, pi: (ni, pi, 0, 0)),
            pl.BlockSpec((1, TPC, 2, C_PTS), lambda ni, pi: (ni, pi, 0, 0)),
            pl.BlockSpec((2, C_PTS), lambda ni, pi: (0, 0)),
            pl.BlockSpec((2, 2 * C_PTS, C), lambda ni, pi: (0, 0, 0)),
            pl.BlockSpec((C,), lambda ni, pi: (0,)),
            pl.BlockSpec((C_PTS,), lambda ni, pi: (0,)),
            pl.BlockSpec((C_PTS,), lambda ni, pi: (0,)),
        ],
        out_specs=[
            pl.BlockSpec((1, TPC, C), lambda ni, pi: (ni, pi, 0)),
            pl.BlockSpec((2, C), lambda ni, pi: (0, 0)),
        ],
        out_shape=[
            jax.ShapeDtypeStruct((NB, P, C), jnp.float32),
            jax.ShapeDtypeStruct((2, C), jnp.float32),
        ],
        scratch_shapes=[pltpu.VMEM((2, C), jnp.float32)],
    )(pooled, gmax, s2, wm, bc1, g2, be2)


def _kd_body(y_ref, s3_ref, w2_ref, bc2_ref, g3_ref, be3_ref, out_ref):
    n3 = float(NB * P)
    m3 = s3_ref[0] / n3
    v3 = s3_ref[1] / n3 - m3 * m3
    sc3 = g3_ref[...] * jax.lax.rsqrt(v3 + EPS)
    sh3 = be3_ref[...] - m3 * sc3
    a = jnp.maximum(y_ref[0] * sc3[None, :] + sh3[None, :], 0.0)
    o = jax.lax.dot_general(w2_ref[...], a, (((1,), (1,)), ((), ())),
                            preferred_element_type=jnp.float32)
    out_ref[0] = o + bc2_ref[...][:, None]


def _kd(y, s3, w2, bc2, g3, be3):
    grid = (NB, P // TPD)
    return pl.pallas_call(
        _kd_body,
        grid=grid,
        in_specs=[
            pl.BlockSpec((1, TPD, C), lambda ni, pi: (ni, pi, 0)),
            pl.BlockSpec((2, C), lambda ni, pi: (0, 0)),
            pl.BlockSpec((C // 4, C), lambda ni, pi: (0, 0)),
            pl.BlockSpec((C // 4,), lambda ni, pi: (0,)),
            pl.BlockSpec((C,), lambda ni, pi: (0,)),
            pl.BlockSpec((C,), lambda ni, pi: (0,)),
        ],
        out_specs=pl.BlockSpec((1, C // 4, TPD), lambda ni, pi: (ni, 0, pi)),
        out_shape=jax.ShapeDtypeStruct((NB, C // 4, P), jnp.float32),
    )(y, s3, w2, bc2, g3, be3)


def kernel(pts, fts_prev, W_fc1, b_fc1, g1, be1, W_fc2, b_fc2, g2, be2, W_c1, b_c1, g3, be3, W_c2, b_c2):
    ptsT = jnp.swapaxes(pts, 1, 2)
    xs = ptsT[:, 0].reshape(-1)
    ys = ptsT[:, 1].reshape(-1)
    zs = ptsT[:, 2].reshape(-1)
    ftsT = jnp.swapaxes(fts_prev, 1, 2).reshape(NB * P, ADD_C)
    ftsT = jnp.pad(ftsT, ((0, 0), (0, 128 - ADD_C)))

    relx, rely, relz, gmax, stats = _sc_stage(xs, ys, zs, ftsT)
    relx = relx.reshape(NB, P, K)
    rely = rely.reshape(NB, P, K)
    relz = relz.reshape(NB, P, K)
    gmax = gmax.reshape(NB, P, 2, C_PTS)
    stats = stats.reshape(NW, 9, 16)

    pooled, s2 = _kb(relx, rely, relz, stats, W_fc1, b_fc1, g1, be1, W_fc2, b_fc2)

    wm = W_c1.reshape(DG, C // DG, 2 * C_PTS // DG, DG)
    wm = jnp.transpose(wm, (0, 3, 2, 1)).reshape(DG, 2 * C_PTS, C // DG)
    wm = jnp.concatenate([
        jnp.concatenate([wm[0], jnp.zeros_like(wm[0])], axis=1)[None],
        jnp.concatenate([jnp.zeros_like(wm[1]), wm[1]], axis=1)[None],
    ], axis=0)

    y, s3 = _kc(pooled, gmax, s2, wm, b_c1, g2, be2)
    out = _kd(y, s3, W_c2, b_c2, g3, be3)
    fts_out = jnp.concatenate([fts_prev, out], axis=1)
    return (pts, fts_out)
